# baseline (speedup 1.0000x reference)
import functools
import jax
import jax.numpy as jnp
from jax import lax
from jax.experimental import pallas as pl
from jax.experimental.pallas import tpu as pltpu
from jax.experimental.pallas import tpu_sc as plsc

_N = 50000
_C = 256
_EPS = 1e-05
_L = 16
_NW = 32
_CHUNK = 200
_NCHUNKS = _N // _CHUNK
_MAXK = (_NCHUNKS + _NW - 1) // _NW
_BR = 2000
_NB = _N // _BR


def _sc_chunk_copy(x_hbm, buf, sem, cid):
    return pltpu.make_async_copy(
        x_hbm.at[pl.ds(cid * _CHUNK, _CHUNK), :], buf, sem)


def _sc_stats_body(x_hbm, out_hbm, buf0, buf1, part, sem0, sem1):
    cidx = lax.axis_index("c")
    sidx = lax.axis_index("s")
    wid = sidx * 2 + cidx

    bufs = (buf0, buf1)
    sems = (sem0, sem1)

    def cid(slot):
        return jnp.minimum(wid + slot * _NW, _NCHUNKS - 1)

    def run_chunk(buf, acc):
        def row_body(r, a):
            s, q = a
            ns, nq = [], []
            for v in range(_L):
                xv = buf[r, pl.ds(v * _L, _L)]
                ns.append(s[v] + xv)
                nq.append(q[v] + xv * xv)
            return (tuple(ns), tuple(nq))
        return lax.fori_loop(0, _CHUNK, row_body, acc)

    zero = jnp.zeros((_L,), jnp.float32)
    acc = (tuple(zero for _ in range(_L)), tuple(zero for _ in range(_L)))

    _sc_chunk_copy(x_hbm, bufs[0], sems[0], cid(0)).start()
    for slot in range(_MAXK):
        b = slot % 2
        nb = (slot + 1) % 2
        if slot + 1 < _MAXK:
            _sc_chunk_copy(x_hbm, bufs[nb], sems[nb], cid(slot + 1)).start()
        _sc_chunk_copy(x_hbm, bufs[b], sems[b], cid(slot)).wait()
        new_s, new_q = run_chunk(bufs[b], acc)
        mv = jnp.broadcast_to(
            (wid + slot * _NW < _NCHUNKS).astype(jnp.float32), (_L,))
        acc = (tuple(o + mv * (n - o) for n, o in zip(new_s, acc[0])),
               tuple(o + mv * (n - o) for n, o in zip(new_q, acc[1])))

    s, q = acc
    for v in range(_L):
        part[0, pl.ds(v * _L, _L)] = s[v]
        part[1, pl.ds(v * _L, _L)] = q[v]
    pltpu.sync_copy(part, out_hbm.at[wid])


_sc_stats_cache = []


def _get_sc_stats():
    if not _sc_stats_cache:
        @functools.partial(
            pl.kernel,
            mesh=plsc.VectorSubcoreMesh(core_axis_name="c",
                                        subcore_axis_name="s"),
            out_type=jax.ShapeDtypeStruct((_NW, 2, _C), jnp.float32),
            scratch_types=[
                pltpu.VMEM((_CHUNK, _C), jnp.float32),
                pltpu.VMEM((_CHUNK, _C), jnp.float32),
                pltpu.VMEM((2, _C), jnp.float32),
                pltpu.SemaphoreType.DMA,
                pltpu.SemaphoreType.DMA,
            ],
        )
        def _sc_stats(x_hbm, out_hbm, buf0, buf1, part, sem0, sem1):
            _sc_stats_body(x_hbm, out_hbm, buf0, buf1, part, sem0, sem1)

        _sc_stats_cache.append(_sc_stats)
    return _sc_stats_cache[0]


def _tc_norm_kernel(x_ref, st_ref, w_ref, b_ref, o_ref):
    ssum = jnp.sum(st_ref[:, 0, :], axis=0, keepdims=True)
    qsum = jnp.sum(st_ref[:, 1, :], axis=0, keepdims=True)
    mean = ssum * (1.0 / _N)
    var = qsum * (1.0 / _N) - mean * mean
    instd = jax.lax.rsqrt(var + _EPS)
    scale = instd * w_ref[:]
    shift = b_ref[:] - mean * scale
    o_ref[:] = x_ref[:] * scale + shift


def kernel(x, weight, bias):
    stats = _get_sc_stats()(x)
    return pl.pallas_call(
        _tc_norm_kernel,
        grid=(_NB,),
        in_specs=[
            pl.BlockSpec((_BR, _C), lambda i: (i, 0)),
            pl.BlockSpec((_NW, 2, _C), lambda i: (0, 0, 0)),
            pl.BlockSpec((1, _C), lambda i: (0, 0)),
            pl.BlockSpec((1, _C), lambda i: (0, 0)),
        ],
        out_specs=pl.BlockSpec((_BR, _C), lambda i: (i, 0)),
        out_shape=jax.ShapeDtypeStruct((_N, _C), jnp.float32),
    )(x, stats, weight, bias)


# speedup vs baseline: 2.3522x; 2.3522x over previous
import jax
import jax.numpy as jnp
from jax.experimental import pallas as pl
from jax.experimental.pallas import tpu as pltpu

_N = 50000
_C = 256
_EPS = 1e-05
_BR = 2000
_NB = _N // _BR
_SUB = 8


def _blk_copy(src, dst, sems, k):
    return pltpu.make_async_copy(
        src.at[pl.ds(k * _BR, _BR), :],
        dst.at[pl.ds(k * _BR, _BR), :],
        sems.at[k],
    )


def _inorm_kernel(x_hbm, w_ref, b_ref, o_hbm, xs_ref, s_ref, q_ref,
                  in_sems, out_sems):
    i = pl.program_id(0)

    @pl.when(i == 0)
    def _start():
        s_ref[:] = jnp.zeros_like(s_ref)
        q_ref[:] = jnp.zeros_like(q_ref)
        for k in range(_NB):
            _blk_copy(x_hbm, xs_ref, in_sems, k).start()

    @pl.when(i < _NB)
    def _accumulate():
        _blk_copy(x_hbm, xs_ref, in_sems, i).wait()
        blk = xs_ref[pl.ds(i * _BR, _BR), :]
        g = blk.reshape(_BR // _SUB, _SUB, _C)
        s_ref[:] += jnp.sum(g, axis=0)
        q_ref[:] += jnp.sum(g * g, axis=0)

    @pl.when(i >= _NB)
    def _normalize():
        j = i - _NB
        ssum = jnp.sum(s_ref[:], axis=0, keepdims=True)
        qsum = jnp.sum(q_ref[:], axis=0, keepdims=True)
        mean = ssum * (1.0 / _N)
        var = qsum * (1.0 / _N) - mean * mean
        instd = jax.lax.rsqrt(var + _EPS)
        scale = instd * w_ref[:]
        shift = b_ref[:] - mean * scale
        xs_ref[pl.ds(j * _BR, _BR), :] = (
            xs_ref[pl.ds(j * _BR, _BR), :] * scale + shift)
        _blk_copy(xs_ref, o_hbm, out_sems, j).start()

    @pl.when(i == 2 * _NB - 1)
    def _drain():
        for k in range(_NB):
            _blk_copy(xs_ref, o_hbm, out_sems, k).wait()


def kernel(x, weight, bias):
    return pl.pallas_call(
        _inorm_kernel,
        grid=(2 * _NB,),
        in_specs=[
            pl.BlockSpec(memory_space=pl.ANY),
            pl.BlockSpec((1, _C), lambda i: (0, 0)),
            pl.BlockSpec((1, _C), lambda i: (0, 0)),
        ],
        out_specs=pl.BlockSpec(memory_space=pl.ANY),
        out_shape=jax.ShapeDtypeStruct((_N, _C), jnp.float32),
        scratch_shapes=[
            pltpu.VMEM((_N, _C), jnp.float32),
            pltpu.VMEM((_SUB, _C), jnp.float32),
            pltpu.VMEM((_SUB, _C), jnp.float32),
            pltpu.SemaphoreType.DMA((_NB,)),
            pltpu.SemaphoreType.DMA((_NB,)),
        ],
    )(x, weight, bias)
